# baseline (speedup 1.0000x reference)
import dataclasses
import functools

import jax
import jax.numpy as jnp
from jax import lax
from jax.experimental import pallas as pl
from jax.experimental.pallas import tpu as pltpu
from jax.experimental.pallas import tpu_sc as plsc

NC = 2
NS = 16
NW = NC * NS
LANES = 16
CHUNK = 128

_MESH = plsc.VectorSubcoreMesh(core_axis_name="c", subcore_axis_name="s")

_GATHER_DNUMS = lax.GatherDimensionNumbers(
    offset_dims=(), collapsed_slice_dims=(0,), start_index_map=(0,))


def _bcast_lane(vec16, r):
    idx = jnp.full((LANES, 1), r, jnp.int32)
    return lax.gather(vec16, idx, _GATHER_DNUMS, (1,),
                      mode=lax.GatherScatterMode.PROMISE_IN_BOUNDS)


_CP_NO_LAYOUT = pltpu.CompilerParams()
if "needs_layout_passes" in pltpu.CompilerParams.__dataclass_fields__:
    _CP_NO_LAYOUT = dataclasses.replace(_CP_NO_LAYOUT,
                                        needs_layout_passes=False)

ECH = 2000


def _sc_degree(col, ew, n):
    e = col.shape[0]
    assert e % ECH == 0
    nchunk = e // ECH
    n_pad = n
    assert n_pad % (LANES * NS) == 0
    npw = n_pad // NS

    @functools.partial(
        pl.kernel,
        out_type=(jax.ShapeDtypeStruct((n_pad,), jnp.float32),
                  jax.ShapeDtypeStruct((n_pad,), jnp.float32)),
        mesh=_MESH,
        scratch_types=[
            pltpu.VMEM((ECH,), jnp.int32),
            pltpu.VMEM((ECH,), jnp.float32),
            pltpu.VMEM((n_pad,), jnp.float32),
            pltpu.VMEM((NS, npw), jnp.float32),
            pltpu.VMEM((npw,), jnp.float32),
            pltpu.VMEM_SHARED((NS, n_pad), jnp.float32),
        ],
        compiler_params=_CP_NO_LAYOUT,
    )
    def deg_kernel(col_hbm, ew_hbm, out0_hbm, out1_hbm,
                   col_v, ew_v, hist, redbuf, res, stage):
        c = lax.axis_index("c")
        s = lax.axis_index("s")
        w = s * NC + c

        @pl.loop(0, n_pad // LANES)
        def _(j):
            hist[pl.ds(j * LANES, LANES)] = jnp.zeros((LANES,), jnp.float32)

        @pl.loop(w, nchunk, step=NW)
        def _(ci):
            base = ci * ECH
            pltpu.sync_copy(col_hbm.at[pl.ds(base, ECH)], col_v)
            pltpu.sync_copy(ew_hbm.at[pl.ds(base, ECH)], ew_v)

            @pl.loop(0, ECH // LANES)
            def _(j):
                i16 = col_v[pl.ds(j * LANES, LANES)]
                v16 = ew_v[pl.ds(j * LANES, LANES)]
                plsc.addupdate_scatter(hist, [i16], v16)

        pltpu.sync_copy(hist, stage.at[s])
        plsc.subcore_barrier()

        @pl.loop(0, NS)
        def _(t):
            pltpu.sync_copy(stage.at[t, pl.ds(s * npw, npw)], redbuf.at[t])

        @pl.loop(0, npw // LANES)
        def _(j):
            sl = pl.ds(j * LANES, LANES)
            acc16 = redbuf[0, sl]
            for t in range(1, NS):
                acc16 = acc16 + redbuf[t, sl]
            res[sl] = acc16

        @pl.when(c == 0)
        def _():
            pltpu.sync_copy(res, out0_hbm.at[pl.ds(s * npw, npw)])

        @pl.when(c == 1)
        def _():
            pltpu.sync_copy(res, out1_hbm.at[pl.ds(s * npw, npw)])

    return deg_kernel(col, ew)


NBM = 2
NBI = 4
ZR = 16


def _sc_aggregate(g, row, col, ew, n):
    e = row.shape[0]
    d = g.shape[1]
    assert e % (CHUNK * NW) == 0 and d % LANES == 0
    nch = e // (CHUNK * NW)
    assert nch % 2 == 0
    wrows = 80
    assert n % wrows == 0 and n % ZR == 0

    @functools.partial(
        pl.kernel,
        out_type=jax.ShapeDtypeStruct((NC, n, d), jnp.float32),
        mesh=_MESH,
        scratch_types=[
            pltpu.VMEM((NBI, CHUNK), jnp.int32),
            pltpu.VMEM((NBI, CHUNK), jnp.int32),
            pltpu.VMEM((NBI, CHUNK), jnp.float32),
            pltpu.VMEM((CHUNK, d), jnp.float32),
            pltpu.VMEM((CHUNK, d), jnp.float32),
            pltpu.VMEM((ZR, d), jnp.float32),
            pltpu.VMEM_SHARED((n, d), jnp.float32),
            pltpu.SemaphoreType.DMA((NBI,)),
            pltpu.SemaphoreType.DMA((NBM,)),
            pltpu.SemaphoreType.DMA((NBM,)),
        ],
    )
    def agg_kernel(g_hbm, row_hbm, col_hbm, ew_hbm, out_hbm,
                   row_v, col_v, ew_v, msg0, msg1, zbuf, acc,
                   idx_sem, gat_sem, sct_sem):
        msgs = (msg0, msg1)
        c = lax.axis_index("c")
        s = lax.axis_index("s")
        w = s * NC + c

        @pl.loop(0, ZR)
        def _(r):
            @pl.loop(0, d // LANES)
            def _(k):
                zbuf[r, pl.ds(k * LANES, LANES)] = jnp.zeros((LANES,),
                                                             jnp.float32)

        @pl.loop(s, n // ZR, step=NS)
        def _(rc):
            pltpu.sync_copy(zbuf, acc.at[pl.ds(rc * ZR, ZR)])

        plsc.subcore_barrier()

        base0 = w * nch

        def start_idx(j):
            b = lax.rem(j, NBI)
            bs = (base0 + j) * CHUNK
            pltpu.async_copy(row_hbm.at[pl.ds(bs, CHUNK)], row_v.at[b],
                             idx_sem.at[b])
            pltpu.async_copy(col_hbm.at[pl.ds(bs, CHUNK)], col_v.at[b],
                             idx_sem.at[b])
            pltpu.async_copy(ew_hbm.at[pl.ds(bs, CHUNK)], ew_v.at[b],
                             idx_sem.at[b])

        def wait_idx(j):
            b = lax.rem(j, NBI)
            pltpu.make_async_copy(row_hbm.at[pl.ds(0, CHUNK)], row_v.at[b],
                                  idx_sem.at[b]).wait()
            pltpu.make_async_copy(col_hbm.at[pl.ds(0, CHUNK)], col_v.at[b],
                                  idx_sem.at[b]).wait()
            pltpu.make_async_copy(ew_hbm.at[pl.ds(0, CHUNK)], ew_v.at[b],
                                  idx_sem.at[b]).wait()

        def start_gather(j, bm):
            bi = lax.rem(j, NBI)
            pltpu.async_copy(g_hbm.at[row_v.at[bi]], msgs[bm],
                             gat_sem.at[bm])

        def wait_gather(j, bm):
            bi = lax.rem(j, NBI)
            pltpu.make_async_copy(g_hbm.at[row_v.at[bi]], msgs[bm],
                                  gat_sem.at[bm]).wait()

        def start_scatter(j, bm):
            bi = lax.rem(j, NBI)
            pltpu.async_copy(msgs[bm], acc.at[col_v.at[bi]],
                             sct_sem.at[bm], add=True)

        def wait_scatter(j, bm):
            bi = lax.rem(j, NBI)
            pltpu.make_async_copy(msgs[bm], acc.at[col_v.at[bi]],
                                  sct_sem.at[bm]).wait()

        start_idx(0)
        start_idx(1)
        wait_idx(0)
        start_gather(0, 0)

        @pl.loop(0, nch // 2)
        def _(t):
            for b in range(2):
                j = t * 2 + b
                wait_gather(j, b)

                @pl.when(j >= 1)
                def _():
                    wait_scatter(j - 1, 1 - b)

                @pl.when(j + 2 < nch)
                def _():
                    start_idx(j + 2)

                @pl.when(j + 1 < nch)
                def _():
                    wait_idx(j + 1)
                    start_gather(j + 1, 1 - b)

                bi = lax.rem(j, NBI)
                msg = msgs[b]

                @pl.loop(0, CHUNK // LANES)
                def _(q):
                    ew16 = ew_v[bi, pl.ds(q * LANES, LANES)]
                    for r in range(LANES):
                        bvec = _bcast_lane(ew16, r)
                        eidx = q * LANES + r
                        for k in range(d // LANES):
                            sl = pl.ds(k * LANES, LANES)
                            msg[eidx, sl] = msg[eidx, sl] * bvec

                start_scatter(j, b)

        wait_scatter(nch - 1, 1)
        plsc.subcore_barrier()

        @pl.loop(s, n // wrows, step=NS)
        def _(rc):
            pltpu.sync_copy(acc.at[pl.ds(rc * wrows, wrows)],
                            out_hbm.at[c, pl.ds(rc * wrows, wrows)])

    return agg_kernel(g, row, col, ew)


def _dis_from_partials(d0_ref, d1_ref, i, blk):
    deg = d0_ref[pl.ds(i * blk, blk)] + d1_ref[pl.ds(i * blk, blk)] + 2.0
    return jnp.where(deg > 0, lax.rsqrt(jnp.where(deg > 0, deg, 1.0)), 0.0)


def _tc_matmul(x, W):
    n, din = x.shape
    dout = W.shape[1]
    blk = 1280
    grid = -(-n // blk)

    def body(x_ref, w_ref, h_ref):
        h_ref[...] = jnp.dot(x_ref[...], w_ref[...],
                             preferred_element_type=jnp.float32)

    return pl.pallas_call(
        body,
        grid=(grid,),
        in_specs=[
            pl.BlockSpec((blk, din), lambda i: (i, 0)),
            pl.BlockSpec((din, dout), lambda i: (0, 0)),
        ],
        out_specs=pl.BlockSpec((blk, dout), lambda i: (i, 0)),
        out_shape=jax.ShapeDtypeStruct((n, dout), jnp.float32),
    )(x, W)


def _tc_transform(h, deg0, deg1):
    n, dout = h.shape
    n_pad = deg0.shape[0]
    blk = 1280
    grid = -(-n // blk)

    def body(h_ref, d0_ref, d1_ref, g_ref):
        i = pl.program_id(0)
        dis = _dis_from_partials(d0_ref, d1_ref, i, blk)
        g_ref[...] = dis[:, None] * h_ref[...]

    return pl.pallas_call(
        body,
        grid=(grid,),
        in_specs=[
            pl.BlockSpec((blk, dout), lambda i: (i, 0)),
            pl.BlockSpec((n_pad,), lambda i: (0,)),
            pl.BlockSpec((n_pad,), lambda i: (0,)),
        ],
        out_specs=pl.BlockSpec((blk, dout), lambda i: (i, 0)),
        out_shape=jax.ShapeDtypeStruct((n, dout), jnp.float32),
    )(h, deg0, deg1)


def _tc_finalize(agg_pp, deg0, deg1, g, b):
    n, dout = g.shape
    n_pad = deg0.shape[0]
    blk = 1280
    grid = -(-n // blk)

    def body(a_ref, d0_ref, d1_ref, g_ref, b_ref, o_ref):
        i = pl.program_id(0)
        dis = _dis_from_partials(d0_ref, d1_ref, i, blk)
        a = a_ref[0] + a_ref[1] + 2.0 * g_ref[...]
        o_ref[...] = jnp.maximum(dis[:, None] * a + b_ref[...], 0.0)

    return pl.pallas_call(
        body,
        grid=(grid,),
        in_specs=[
            pl.BlockSpec((NC, blk, dout), lambda i: (0, i, 0)),
            pl.BlockSpec((n_pad,), lambda i: (0,)),
            pl.BlockSpec((n_pad,), lambda i: (0,)),
            pl.BlockSpec((blk, dout), lambda i: (i, 0)),
            pl.BlockSpec((1, dout), lambda i: (0, 0)),
        ],
        out_specs=pl.BlockSpec((blk, dout), lambda i: (i, 0)),
        out_shape=jax.ShapeDtypeStruct((n, dout), jnp.float32),
    )(agg_pp, deg0, deg1, g, b.reshape(1, dout))


def kernel(x, edge_index, edge_weight, W, b):
    n = x.shape[0]
    n_pad = -(-n // 1280) * 1280
    row = edge_index[0]
    col = edge_index[1]
    e = row.shape[0]
    e_pad = -(-e // (CHUNK * NW * 2)) * (CHUNK * NW * 2)
    fill = jnp.arange(e_pad - e, dtype=row.dtype) % n
    row_p = jnp.concatenate([row, fill])
    col_p = jnp.concatenate([col, fill])
    ew_p = jnp.concatenate([edge_weight,
                            jnp.zeros((e_pad - e,), edge_weight.dtype)])
    h = _tc_matmul(x, W)
    deg0, deg1 = _sc_degree(col, edge_weight, n_pad)
    g = _tc_transform(h, deg0, deg1)
    agg_pp = _sc_aggregate(g, row_p, col_p, ew_p, n)
    return _tc_finalize(agg_pp, deg0, deg1, g, b)


# speedup vs baseline: 1.0197x; 1.0197x over previous
import dataclasses
import functools

import jax
import jax.numpy as jnp
from jax import lax
from jax.experimental import pallas as pl
from jax.experimental.pallas import tpu as pltpu
from jax.experimental.pallas import tpu_sc as plsc

NC = 2
NS = 16
NW = NC * NS
LANES = 16
CHUNK = 128

_MESH = plsc.VectorSubcoreMesh(core_axis_name="c", subcore_axis_name="s")

_GATHER_DNUMS = lax.GatherDimensionNumbers(
    offset_dims=(), collapsed_slice_dims=(0,), start_index_map=(0,))


def _bcast_lane(vec16, r):
    idx = jnp.full((LANES, 1), r, jnp.int32)
    return lax.gather(vec16, idx, _GATHER_DNUMS, (1,),
                      mode=lax.GatherScatterMode.PROMISE_IN_BOUNDS)


_CP_NO_LAYOUT = pltpu.CompilerParams()
if "needs_layout_passes" in pltpu.CompilerParams.__dataclass_fields__:
    _CP_NO_LAYOUT = dataclasses.replace(_CP_NO_LAYOUT,
                                        needs_layout_passes=False)

ECH = 2000


def _sc_degree(col, ew, n):
    e = col.shape[0]
    assert e % ECH == 0
    nchunk = e // ECH
    n_pad = n
    assert n_pad % (LANES * NS) == 0
    npw = n_pad // NS

    @functools.partial(
        pl.kernel,
        out_type=(jax.ShapeDtypeStruct((n_pad,), jnp.float32),
                  jax.ShapeDtypeStruct((n_pad,), jnp.float32)),
        mesh=_MESH,
        scratch_types=[
            pltpu.VMEM((ECH,), jnp.int32),
            pltpu.VMEM((ECH,), jnp.float32),
            pltpu.VMEM((n_pad,), jnp.float32),
            pltpu.VMEM((NS, npw), jnp.float32),
            pltpu.VMEM((npw,), jnp.float32),
            pltpu.VMEM_SHARED((NS, n_pad), jnp.float32),
            pltpu.SemaphoreType.DMA,
        ],
        compiler_params=_CP_NO_LAYOUT,
    )
    def deg_kernel(col_hbm, ew_hbm, out0_hbm, out1_hbm,
                   col_v, ew_v, hist, redbuf, res, stage, red_sem):
        c = lax.axis_index("c")
        s = lax.axis_index("s")
        w = s * NC + c

        @pl.loop(0, n_pad // LANES)
        def _(j):
            hist[pl.ds(j * LANES, LANES)] = jnp.zeros((LANES,), jnp.float32)

        @pl.loop(w, nchunk, step=NW)
        def _(ci):
            base = ci * ECH
            pltpu.sync_copy(col_hbm.at[pl.ds(base, ECH)], col_v)
            pltpu.sync_copy(ew_hbm.at[pl.ds(base, ECH)], ew_v)

            @pl.loop(0, ECH // LANES)
            def _(j):
                i16 = col_v[pl.ds(j * LANES, LANES)]
                v16 = ew_v[pl.ds(j * LANES, LANES)]
                plsc.addupdate_scatter(hist, [i16], v16)

        pltpu.sync_copy(hist, stage.at[s])
        plsc.subcore_barrier()

        @pl.loop(0, NS)
        def _(t):
            pltpu.async_copy(stage.at[t, pl.ds(s * npw, npw)], redbuf.at[t],
                             red_sem)

        @pl.loop(0, NS)
        def _(t):
            pltpu.make_async_copy(stage.at[t, pl.ds(s * npw, npw)],
                                  redbuf.at[t], red_sem).wait()

        @pl.loop(0, npw // LANES)
        def _(j):
            sl = pl.ds(j * LANES, LANES)
            acc16 = redbuf[0, sl]
            for t in range(1, NS):
                acc16 = acc16 + redbuf[t, sl]
            res[sl] = acc16

        @pl.when(c == 0)
        def _():
            pltpu.sync_copy(res, out0_hbm.at[pl.ds(s * npw, npw)])

        @pl.when(c == 1)
        def _():
            pltpu.sync_copy(res, out1_hbm.at[pl.ds(s * npw, npw)])

    return deg_kernel(col, ew)


NBM = 2
NBI = 4
ZR = 80


def _sc_aggregate(g, row, col, ew, n):
    e = row.shape[0]
    d = g.shape[1]
    assert e % (CHUNK * NW) == 0 and d % LANES == 0
    nch = e // (CHUNK * NW)
    assert nch % 2 == 0
    assert n % ZR == 0
    nz = n // ZR

    @functools.partial(
        pl.kernel,
        out_type=jax.ShapeDtypeStruct((NC, n, d), jnp.float32),
        mesh=_MESH,
        scratch_types=[
            pltpu.VMEM((NBI, CHUNK), jnp.int32),
            pltpu.VMEM((NBI, CHUNK), jnp.int32),
            pltpu.VMEM((NBI, CHUNK), jnp.float32),
            pltpu.VMEM((CHUNK, d), jnp.float32),
            pltpu.VMEM((CHUNK, d), jnp.float32),
            pltpu.VMEM((ZR, d), jnp.float32),
            pltpu.VMEM_SHARED((n, d), jnp.float32),
            pltpu.SemaphoreType.DMA((NBI,)),
            pltpu.SemaphoreType.DMA((NBM,)),
            pltpu.SemaphoreType.DMA((NBM,)),
            pltpu.SemaphoreType.DMA,
        ],
    )
    def agg_kernel(g_hbm, row_hbm, col_hbm, ew_hbm, out_hbm,
                   row_v, col_v, ew_v, msg0, msg1, zbuf, acc,
                   idx_sem, gat_sem, sct_sem, z_sem):
        msgs = (msg0, msg1)
        c = lax.axis_index("c")
        s = lax.axis_index("s")
        w = s * NC + c

        @pl.loop(0, ZR)
        def _(r):
            @pl.loop(0, d // LANES)
            def _(k):
                zbuf[r, pl.ds(k * LANES, LANES)] = jnp.zeros((LANES,),
                                                             jnp.float32)

        @pl.loop(s, nz, step=NS)
        def _(rc):
            pltpu.async_copy(zbuf, acc.at[pl.ds(rc * ZR, ZR)], z_sem)

        @pl.loop(s, nz, step=NS)
        def _(rc):
            pltpu.make_async_copy(zbuf, acc.at[pl.ds(rc * ZR, ZR)],
                                  z_sem).wait()

        plsc.subcore_barrier()

        base0 = w * nch

        def start_idx(j):
            b = lax.rem(j, NBI)
            bs = (base0 + j) * CHUNK
            pltpu.async_copy(row_hbm.at[pl.ds(bs, CHUNK)], row_v.at[b],
                             idx_sem.at[b])
            pltpu.async_copy(col_hbm.at[pl.ds(bs, CHUNK)], col_v.at[b],
                             idx_sem.at[b])
            pltpu.async_copy(ew_hbm.at[pl.ds(bs, CHUNK)], ew_v.at[b],
                             idx_sem.at[b])

        def wait_idx(j):
            b = lax.rem(j, NBI)
            pltpu.make_async_copy(row_hbm.at[pl.ds(0, CHUNK)], row_v.at[b],
                                  idx_sem.at[b]).wait()
            pltpu.make_async_copy(col_hbm.at[pl.ds(0, CHUNK)], col_v.at[b],
                                  idx_sem.at[b]).wait()
            pltpu.make_async_copy(ew_hbm.at[pl.ds(0, CHUNK)], ew_v.at[b],
                                  idx_sem.at[b]).wait()

        def start_gather(j, bm):
            bi = lax.rem(j, NBI)
            pltpu.async_copy(g_hbm.at[row_v.at[bi]], msgs[bm],
                             gat_sem.at[bm])

        def wait_gather(j, bm):
            bi = lax.rem(j, NBI)
            pltpu.make_async_copy(g_hbm.at[row_v.at[bi]], msgs[bm],
                                  gat_sem.at[bm]).wait()

        def start_scatter(j, bm):
            bi = lax.rem(j, NBI)
            pltpu.async_copy(msgs[bm], acc.at[col_v.at[bi]],
                             sct_sem.at[bm], add=True)

        def wait_scatter(j, bm):
            bi = lax.rem(j, NBI)
            pltpu.make_async_copy(msgs[bm], acc.at[col_v.at[bi]],
                                  sct_sem.at[bm]).wait()

        start_idx(0)
        start_idx(1)
        wait_idx(0)
        start_gather(0, 0)

        @pl.loop(0, nch // 2)
        def _(t):
            for b in range(2):
                j = t * 2 + b
                wait_gather(j, b)

                @pl.when(j >= 1)
                def _():
                    wait_scatter(j - 1, 1 - b)

                @pl.when(j + 2 < nch)
                def _():
                    start_idx(j + 2)

                @pl.when(j + 1 < nch)
                def _():
                    wait_idx(j + 1)
                    start_gather(j + 1, 1 - b)

                bi = lax.rem(j, NBI)
                msg = msgs[b]

                @pl.loop(0, CHUNK // LANES)
                def _(q):
                    ew16 = ew_v[bi, pl.ds(q * LANES, LANES)]
                    for r in range(LANES):
                        bvec = _bcast_lane(ew16, r)
                        eidx = q * LANES + r
                        for k in range(d // LANES):
                            sl = pl.ds(k * LANES, LANES)
                            msg[eidx, sl] = msg[eidx, sl] * bvec

                start_scatter(j, b)

        wait_scatter(nch - 1, 1)
        plsc.subcore_barrier()

        @pl.loop(s, nz, step=NS)
        def _(rc):
            pltpu.async_copy(acc.at[pl.ds(rc * ZR, ZR)],
                             out_hbm.at[c, pl.ds(rc * ZR, ZR)], z_sem)

        @pl.loop(s, nz, step=NS)
        def _(rc):
            pltpu.make_async_copy(acc.at[pl.ds(rc * ZR, ZR)],
                                  out_hbm.at[c, pl.ds(rc * ZR, ZR)],
                                  z_sem).wait()

    return agg_kernel(g, row, col, ew)


def _dis_from_partials(d0_ref, d1_ref, i, blk):
    deg = d0_ref[pl.ds(i * blk, blk)] + d1_ref[pl.ds(i * blk, blk)] + 2.0
    return jnp.where(deg > 0, lax.rsqrt(jnp.where(deg > 0, deg, 1.0)), 0.0)


def _tc_matmul(x, W):
    n, din = x.shape
    dout = W.shape[1]
    blk = 1280
    grid = -(-n // blk)

    def body(x_ref, w_ref, h_ref):
        h_ref[...] = jnp.dot(x_ref[...], w_ref[...],
                             preferred_element_type=jnp.float32)

    return pl.pallas_call(
        body,
        grid=(grid,),
        in_specs=[
            pl.BlockSpec((blk, din), lambda i: (i, 0)),
            pl.BlockSpec((din, dout), lambda i: (0, 0)),
        ],
        out_specs=pl.BlockSpec((blk, dout), lambda i: (i, 0)),
        out_shape=jax.ShapeDtypeStruct((n, dout), jnp.float32),
    )(x, W)


def _tc_transform(h, deg0, deg1):
    n, dout = h.shape
    n_pad = deg0.shape[0]
    blk = 1280
    grid = -(-n // blk)

    def body(h_ref, d0_ref, d1_ref, g_ref):
        i = pl.program_id(0)
        dis = _dis_from_partials(d0_ref, d1_ref, i, blk)
        g_ref[...] = dis[:, None] * h_ref[...]

    return pl.pallas_call(
        body,
        grid=(grid,),
        in_specs=[
            pl.BlockSpec((blk, dout), lambda i: (i, 0)),
            pl.BlockSpec((n_pad,), lambda i: (0,)),
            pl.BlockSpec((n_pad,), lambda i: (0,)),
        ],
        out_specs=pl.BlockSpec((blk, dout), lambda i: (i, 0)),
        out_shape=jax.ShapeDtypeStruct((n, dout), jnp.float32),
    )(h, deg0, deg1)


def _tc_finalize(agg_pp, deg0, deg1, g, b):
    n, dout = g.shape
    n_pad = deg0.shape[0]
    blk = 1280
    grid = -(-n // blk)

    def body(a_ref, d0_ref, d1_ref, g_ref, b_ref, o_ref):
        i = pl.program_id(0)
        dis = _dis_from_partials(d0_ref, d1_ref, i, blk)
        a = a_ref[0] + a_ref[1] + 2.0 * g_ref[...]
        o_ref[...] = jnp.maximum(dis[:, None] * a + b_ref[...], 0.0)

    return pl.pallas_call(
        body,
        grid=(grid,),
        in_specs=[
            pl.BlockSpec((NC, blk, dout), lambda i: (0, i, 0)),
            pl.BlockSpec((n_pad,), lambda i: (0,)),
            pl.BlockSpec((n_pad,), lambda i: (0,)),
            pl.BlockSpec((blk, dout), lambda i: (i, 0)),
            pl.BlockSpec((1, dout), lambda i: (0, 0)),
        ],
        out_specs=pl.BlockSpec((blk, dout), lambda i: (i, 0)),
        out_shape=jax.ShapeDtypeStruct((n, dout), jnp.float32),
    )(agg_pp, deg0, deg1, g, b.reshape(1, dout))


def kernel(x, edge_index, edge_weight, W, b):
    n = x.shape[0]
    n_pad = -(-n // 1280) * 1280
    row = edge_index[0]
    col = edge_index[1]
    e = row.shape[0]
    e_pad = -(-e // (CHUNK * NW * 2)) * (CHUNK * NW * 2)
    fill = jnp.arange(e_pad - e, dtype=row.dtype) % n
    row_p = jnp.concatenate([row, fill])
    col_p = jnp.concatenate([col, fill])
    ew_p = jnp.concatenate([edge_weight,
                            jnp.zeros((e_pad - e,), edge_weight.dtype)])
    h = _tc_matmul(x, W)
    deg0, deg1 = _sc_degree(col, edge_weight, n_pad)
    g = _tc_transform(h, deg0, deg1)
    agg_pp = _sc_aggregate(g, row_p, col_p, ew_p, n)
    return _tc_finalize(agg_pp, deg0, deg1, g, b)
